# Initial kernel scaffold; baseline (speedup 1.0000x reference)
import jax
import jax.numpy as jnp
from jax.experimental import pallas as pl


def kernel(X, mask, residue_idx, chain_labels, W_pos, b_pos, W_edge, ln_scale, ln_bias):
    raise NotImplementedError("write your pallas kernel here")


# speedup vs baseline: 1.1503x; 1.1503x over previous
import functools

import jax
import jax.numpy as jnp
from jax import lax
from jax.experimental import pallas as pl
from jax.experimental.pallas import tpu as pltpu
from jax.experimental.pallas import tpu_sc as plsc

K = 30
NRBF = 16
MAXREL = 32
DMIN, DMAX = 2.0, 22.0
POS_PAD = 72
ROWF = 16

SC_NC, SC_NS = 2, 16
SC_NW = SC_NC * SC_NS


def _prep_body(x_ref, resi_ref, chain_ref, wp_ref, bp_ref, we_ref,
               table_ref, t_ref):
    x = x_ref[...]
    n = x[:, 0:3]
    c = x[:, 3:6]
    ca = x[:, 6:9]
    bvec = ca - n
    cvec = c - ca
    ax = bvec[:, 1:2] * cvec[:, 2:3] - bvec[:, 2:3] * cvec[:, 1:2]
    ay = bvec[:, 2:3] * cvec[:, 0:1] - bvec[:, 0:1] * cvec[:, 2:3]
    az = bvec[:, 0:1] * cvec[:, 1:2] - bvec[:, 1:2] * cvec[:, 0:1]
    a = jnp.concatenate([ax, ay, az], axis=1)
    cb = -0.58273431 * a + 0.56802827 * bvec - 0.54067466 * cvec + ca
    resi = resi_ref[...]
    chain = chain_ref[...]
    zero = jnp.zeros_like(resi)
    table_ref[...] = jnp.concatenate([n, c, ca, cb, resi, chain, zero, zero],
                                     axis=1)
    wp = wp_ref[...] + bp_ref[...]
    t_ref[...] = jnp.dot(wp, we_ref[0:16, :], preferred_element_type=jnp.float32)


def _topk_body(qc_ref, kc_ref, idx_ref, fidx_ref, *, L, TQ):
    b = pl.program_id(0)
    qx = qc_ref[0, :, 0:1]
    qy = qc_ref[0, :, 1:2]
    qz = qc_ref[0, :, 2:3]
    kx = kc_ref[0, 0:1, :]
    ky = kc_ref[0, 1:2, :]
    kz = kc_ref[0, 2:3, :]
    dx = qx - kx
    dy = qy - ky
    dz = qz - kz
    d2 = dx * dx + dy * dy + dz * dz + 1e-6
    iota = lax.broadcasted_iota(jnp.int32, (TQ, L), 1)
    cols = []
    for j in range(K):
        m = jnp.min(d2, axis=1, keepdims=True)
        cand = jnp.where(d2 == m, iota, L)
        idx = jnp.min(cand, axis=1, keepdims=True)
        cols.append(idx)
        if j < K - 1:
            d2 = jnp.where(iota == idx, jnp.inf, d2)
    idxm = jnp.concatenate(cols, axis=1)
    idx_ref[0] = idxm
    fidx_ref[0] = idxm + b * L


def _sc_gather_body(table_hbm, cidx_hbm, out_hbm, idx_v, rows_v, sem,
                    *, n_chunks, chunk):
    wid = lax.axis_index("s") * SC_NC + lax.axis_index("c")
    base = wid * (n_chunks * chunk)
    for ci in range(n_chunks):
        off = base + ci * chunk
        pltpu.sync_copy(cidx_hbm.at[pl.ds(off, chunk)], idx_v)
        pltpu.async_copy(table_hbm.at[idx_v], rows_v, sem).wait()
        pltpu.sync_copy(rows_v, out_hbm.at[pl.ds(off, chunk)])


_PAIRS = [(3, 3), (0, 0), (6, 6), (9, 9), (3, 0), (3, 6), (3, 9), (0, 6),
          (0, 9), (9, 6), (0, 3), (6, 3), (9, 3), (6, 0), (9, 0), (6, 9)]


def _edges_body(gq_ref, gn_ref, t_ref, w2_ref, s_ref, b_ref,
                out_ref, *, EB):
    q = gq_ref[...]
    n = gn_ref[...]
    mu = DMIN + lax.broadcasted_iota(jnp.int32, (1, NRBF), 1).astype(
        jnp.float32) * ((DMAX - DMIN) / (NRBF - 1))
    sigma = (DMAX - DMIN) / NRBF

    def rbf(d):
        z = (d - mu) / sigma
        return jnp.exp(-(z * z))

    feats = []
    for pq, pn in _PAIRS:
        dx = q[:, pq:pq + 1] - n[:, pn:pn + 1]
        dy = q[:, pq + 1:pq + 2] - n[:, pn + 1:pn + 2]
        dz = q[:, pq + 2:pq + 3] - n[:, pn + 2:pn + 3]
        feats.append(rbf(jnp.sqrt(dx * dx + dy * dy + dz * dz + 1e-6)))
    f = jnp.concatenate(feats, axis=1)
    e = jnp.dot(f, w2_ref[...], preferred_element_type=jnp.float32)

    off = q[:, 12:13] - n[:, 12:13]
    same = q[:, 13:14] == n[:, 13:14]
    d = jnp.where(same, jnp.clip(off + float(MAXREL), 0.0, 2.0 * MAXREL),
                  2.0 * MAXREL + 1.0).astype(jnp.int32)
    onehot = (lax.broadcasted_iota(jnp.int32, (EB, POS_PAD), 1) == d
              ).astype(jnp.float32)
    e = e + jnp.dot(onehot, t_ref[...], preferred_element_type=jnp.float32)

    m = jnp.mean(e, axis=1, keepdims=True)
    xc = e - m
    var = jnp.mean(xc * xc, axis=1, keepdims=True)
    out_ref[...] = xc * lax.rsqrt(var + 1e-5) * s_ref[...] + b_ref[...]


def kernel(X, mask, residue_idx, chain_labels, W_pos, b_pos, W_edge,
           ln_scale, ln_bias):
    B, L = X.shape[0], X.shape[1]
    R = B * L
    E = R * K
    f32 = jnp.float32

    Xr = X.reshape(R, 12)
    resi_f = residue_idx.reshape(R, 1).astype(f32)
    chain_f = chain_labels.reshape(R, 1).astype(f32)
    wp_pad = jnp.zeros((POS_PAD, W_pos.shape[1]), f32).at[:W_pos.shape[0]].set(W_pos)
    bp = b_pos.reshape(1, -1)
    table, T = pl.pallas_call(
        _prep_body,
        out_shape=(jax.ShapeDtypeStruct((R, ROWF), f32),
                   jax.ShapeDtypeStruct((POS_PAD, 128), f32)),
    )(Xr, resi_f, chain_f, wp_pad, bp, W_edge)

    TQ = 256
    Cq = X[:, :, 1, :]
    qc = jnp.concatenate([Cq, jnp.zeros((B, L, 1), f32)], axis=2)
    kc = jnp.swapaxes(qc, 1, 2)
    i32 = jnp.int32
    E_idx, fidx = pl.pallas_call(
        functools.partial(_topk_body, L=L, TQ=TQ),
        grid=(B, L // TQ),
        in_specs=[
            pl.BlockSpec((1, TQ, 4), lambda b, q: (b, q, 0)),
            pl.BlockSpec((1, 4, L), lambda b, q: (b, 0, 0)),
        ],
        out_specs=[
            pl.BlockSpec((1, TQ, K), lambda b, q: (b, q, 0)),
            pl.BlockSpec((1, TQ, K), lambda b, q: (b, q, 0)),
        ],
        out_shape=[
            jax.ShapeDtypeStruct((B, L, K), i32),
            jax.ShapeDtypeStruct((B, L, K), i32),
        ],
    )(qc, kc)

    qidx = jnp.repeat(jnp.arange(R, dtype=i32), K)
    cidx = jnp.concatenate([qidx, fidx.reshape(E)])
    per_w = (2 * E) // SC_NW
    chunk = 1920
    n_chunks = per_w // chunk
    mesh = plsc.VectorSubcoreMesh(core_axis_name="c", subcore_axis_name="s")
    gath = pl.kernel(
        functools.partial(_sc_gather_body, n_chunks=n_chunks, chunk=chunk),
        out_type=jax.ShapeDtypeStruct((2 * E, ROWF), f32),
        mesh=mesh,
        compiler_params=pltpu.CompilerParams(use_tc_tiling_on_sc=False),
        scratch_types=[
            pltpu.VMEM((chunk,), i32),
            pltpu.VMEM((chunk, ROWF), f32),
            pltpu.SemaphoreType.DMA,
        ],
    )(table, cidx)
    gq = gath[:E]
    gn = gath[E:]

    EB = 3840
    W2 = W_edge[NRBF:, :]
    Eout = pl.pallas_call(
        functools.partial(_edges_body, EB=EB),
        grid=(E // EB,),
        in_specs=[
            pl.BlockSpec((EB, ROWF), lambda i: (i, 0)),
            pl.BlockSpec((EB, ROWF), lambda i: (i, 0)),
            pl.BlockSpec((POS_PAD, 128), lambda i: (0, 0)),
            pl.BlockSpec((W2.shape[0], 128), lambda i: (0, 0)),
            pl.BlockSpec((1, 128), lambda i: (0, 0)),
            pl.BlockSpec((1, 128), lambda i: (0, 0)),
        ],
        out_specs=pl.BlockSpec((EB, 128), lambda i: (i, 0)),
        out_shape=jax.ShapeDtypeStruct((E, 128), f32),
    )(gq, gn, T, W2, ln_scale.reshape(1, 128), ln_bias.reshape(1, 128))

    return (Eout.reshape(B, L, K, 128), E_idx)


# speedup vs baseline: 2.5743x; 2.2379x over previous
import functools

import jax
import jax.numpy as jnp
from jax import lax
from jax.experimental import pallas as pl
from jax.experimental.pallas import tpu as pltpu
from jax.experimental.pallas import tpu_sc as plsc

K = 30
NRBF = 16
MAXREL = 32
DMIN, DMAX = 2.0, 22.0
POS_PAD = 72
ROWF = 16

SC_NC, SC_NS = 2, 16
SC_NW = SC_NC * SC_NS


def _prep_body(x_ref, resi_ref, chain_ref, wp_ref, bp_ref, we_ref,
               table_ref, t_ref):
    x = x_ref[...]
    n = x[:, 0:3]
    c = x[:, 3:6]
    ca = x[:, 6:9]
    bvec = ca - n
    cvec = c - ca
    ax = bvec[:, 1:2] * cvec[:, 2:3] - bvec[:, 2:3] * cvec[:, 1:2]
    ay = bvec[:, 2:3] * cvec[:, 0:1] - bvec[:, 0:1] * cvec[:, 2:3]
    az = bvec[:, 0:1] * cvec[:, 1:2] - bvec[:, 1:2] * cvec[:, 0:1]
    a = jnp.concatenate([ax, ay, az], axis=1)
    cb = -0.58273431 * a + 0.56802827 * bvec - 0.54067466 * cvec + ca
    resi = resi_ref[...]
    chain = chain_ref[...]
    zero = jnp.zeros_like(resi)
    table_ref[...] = jnp.concatenate([n, c, ca, cb, resi, chain, zero, zero],
                                     axis=1)
    wp = wp_ref[...] + bp_ref[...]
    t_ref[...] = jnp.dot(wp, we_ref[0:16, :], preferred_element_type=jnp.float32)


def _topk_body(qc_ref, kc_ref, idx_ref, fidx_ref, *, L, TQ):
    b = pl.program_id(0)
    qx = qc_ref[0, :, 0:1]
    qy = qc_ref[0, :, 1:2]
    qz = qc_ref[0, :, 2:3]
    kx = kc_ref[0, 0:1, :]
    ky = kc_ref[0, 1:2, :]
    kz = kc_ref[0, 2:3, :]
    dx = qx - kx
    dy = qy - ky
    dz = qz - kz
    d2 = dx * dx + dy * dy + dz * dz + 1e-6
    iota = lax.broadcasted_iota(jnp.int32, (TQ, L), 1)
    cols = []
    for j in range(K):
        m = jnp.min(d2, axis=1, keepdims=True)
        cand = jnp.where(d2 == m, iota, L)
        idx = jnp.min(cand, axis=1, keepdims=True)
        cols.append(idx)
        if j < K - 1:
            d2 = jnp.where(iota == idx, jnp.inf, d2)
    idxm = jnp.concatenate(cols, axis=1)
    idx_ref[0] = idxm
    fidx_ref[0] = idxm + b * L


def _sc_gather_body(table_hbm, cidx_hbm, out_hbm, idx_v, rows_v, sem,
                    *, n_chunks, chunk):
    wid = lax.axis_index("s") * SC_NC + lax.axis_index("c")
    base = wid * (n_chunks * chunk)
    for ci in range(n_chunks):
        off = base + ci * chunk
        pltpu.sync_copy(cidx_hbm.at[pl.ds(off, chunk)], idx_v)
        pltpu.async_copy(table_hbm.at[idx_v], rows_v, sem).wait()
        pltpu.sync_copy(rows_v, out_hbm.at[pl.ds(off, chunk)])


_PAIRS = [(3, 3), (0, 0), (6, 6), (9, 9), (3, 0), (3, 6), (3, 9), (0, 6),
          (0, 9), (9, 6), (0, 3), (6, 3), (9, 3), (6, 0), (9, 0), (6, 9)]


def _edges_body(gq_ref, gn_ref, t_ref, w2_ref, s_ref, b_ref,
                out_ref, *, EB):
    q = gq_ref[...]
    n = gn_ref[...]
    mu = DMIN + lax.broadcasted_iota(jnp.int32, (NRBF, 1), 0).astype(
        jnp.float32) * ((DMAX - DMIN) / (NRBF - 1))
    sigma = (DMAX - DMIN) / NRBF

    def rbf(d):
        z = (d - mu) / sigma
        return jnp.exp(-(z * z))

    feats = []
    for pq, pn in _PAIRS:
        dx = q[pq:pq + 1, :] - n[pn:pn + 1, :]
        dy = q[pq + 1:pq + 2, :] - n[pn + 1:pn + 2, :]
        dz = q[pq + 2:pq + 3, :] - n[pn + 2:pn + 3, :]
        feats.append(rbf(jnp.sqrt(dx * dx + dy * dy + dz * dz + 1e-6)))
    f = jnp.concatenate(feats, axis=0)
    dims = (((0,), (0,)), ((), ()))
    e = lax.dot_general(f, w2_ref[...], dims,
                        preferred_element_type=jnp.float32)

    off = q[12:13, :] - n[12:13, :]
    same = q[13:14, :] == n[13:14, :]
    d = jnp.where(same, jnp.clip(off + float(MAXREL), 0.0, 2.0 * MAXREL),
                  2.0 * MAXREL + 1.0).astype(jnp.int32)
    onehot = (lax.broadcasted_iota(jnp.int32, (POS_PAD, EB), 0) == d
              ).astype(jnp.float32)
    e = e + lax.dot_general(onehot, t_ref[...], dims,
                            preferred_element_type=jnp.float32)

    m = jnp.mean(e, axis=1, keepdims=True)
    xc = e - m
    var = jnp.mean(xc * xc, axis=1, keepdims=True)
    out_ref[...] = xc * lax.rsqrt(var + 1e-5) * s_ref[...] + b_ref[...]


def kernel(X, mask, residue_idx, chain_labels, W_pos, b_pos, W_edge,
           ln_scale, ln_bias):
    B, L = X.shape[0], X.shape[1]
    R = B * L
    E = R * K
    f32 = jnp.float32

    Xr = X.reshape(R, 12)
    resi_f = residue_idx.reshape(R, 1).astype(f32)
    chain_f = chain_labels.reshape(R, 1).astype(f32)
    wp_pad = jnp.zeros((POS_PAD, W_pos.shape[1]), f32).at[:W_pos.shape[0]].set(W_pos)
    bp = b_pos.reshape(1, -1)
    table, T = pl.pallas_call(
        _prep_body,
        out_shape=(jax.ShapeDtypeStruct((R, ROWF), f32),
                   jax.ShapeDtypeStruct((POS_PAD, 128), f32)),
    )(Xr, resi_f, chain_f, wp_pad, bp, W_edge)

    TQ = 256
    Cq = X[:, :, 1, :]
    qc = jnp.concatenate([Cq, jnp.zeros((B, L, 1), f32)], axis=2)
    kc = jnp.swapaxes(qc, 1, 2)
    i32 = jnp.int32
    E_idx, fidx = pl.pallas_call(
        functools.partial(_topk_body, L=L, TQ=TQ),
        grid=(B, L // TQ),
        in_specs=[
            pl.BlockSpec((1, TQ, 4), lambda b, q: (b, q, 0)),
            pl.BlockSpec((1, 4, L), lambda b, q: (b, 0, 0)),
        ],
        out_specs=[
            pl.BlockSpec((1, TQ, K), lambda b, q: (b, q, 0)),
            pl.BlockSpec((1, TQ, K), lambda b, q: (b, q, 0)),
        ],
        out_shape=[
            jax.ShapeDtypeStruct((B, L, K), i32),
            jax.ShapeDtypeStruct((B, L, K), i32),
        ],
    )(qc, kc)

    qidx = jnp.repeat(jnp.arange(R, dtype=i32), K)
    cidx = jnp.concatenate([qidx, fidx.reshape(E)])
    per_w = (2 * E) // SC_NW
    chunk = 1920
    n_chunks = per_w // chunk
    mesh = plsc.VectorSubcoreMesh(core_axis_name="c", subcore_axis_name="s")
    gath = pl.kernel(
        functools.partial(_sc_gather_body, n_chunks=n_chunks, chunk=chunk),
        out_type=jax.ShapeDtypeStruct((2 * E, ROWF), f32),
        mesh=mesh,
        compiler_params=pltpu.CompilerParams(use_tc_tiling_on_sc=False),
        scratch_types=[
            pltpu.VMEM((chunk,), i32),
            pltpu.VMEM((chunk, ROWF), f32),
            pltpu.SemaphoreType.DMA,
        ],
    )(table, cidx)
    gathT = gath.T
    gqT = gathT[:, :E]
    gnT = gathT[:, E:]

    EB = 3840
    W2 = W_edge[NRBF:, :]
    Eout = pl.pallas_call(
        functools.partial(_edges_body, EB=EB),
        grid=(E // EB,),
        in_specs=[
            pl.BlockSpec((ROWF, EB), lambda i: (0, i)),
            pl.BlockSpec((ROWF, EB), lambda i: (0, i)),
            pl.BlockSpec((POS_PAD, 128), lambda i: (0, 0)),
            pl.BlockSpec((W2.shape[0], 128), lambda i: (0, 0)),
            pl.BlockSpec((1, 128), lambda i: (0, 0)),
            pl.BlockSpec((1, 128), lambda i: (0, 0)),
        ],
        out_specs=pl.BlockSpec((EB, 128), lambda i: (i, 0)),
        out_shape=jax.ShapeDtypeStruct((E, 128), f32),
    )(gqT, gnT, T, W2, ln_scale.reshape(1, 128), ln_bias.reshape(1, 128))

    return (Eout.reshape(B, L, K, 128), E_idx)


# speedup vs baseline: 2.7885x; 1.0832x over previous
import functools

import jax
import jax.numpy as jnp
from jax import lax
from jax.experimental import pallas as pl
from jax.experimental.pallas import tpu as pltpu
from jax.experimental.pallas import tpu_sc as plsc

K = 30
NRBF = 16
MAXREL = 32
DMIN, DMAX = 2.0, 22.0
POS_PAD = 72
ROWF = 16

SC_NC, SC_NS = 2, 16
SC_NW = SC_NC * SC_NS


def _prep_body(x_ref, resi_ref, chain_ref, wp_ref, bp_ref, we_ref,
               table_ref, t_ref):
    x = x_ref[...]
    n = x[:, 0:3]
    c = x[:, 3:6]
    ca = x[:, 6:9]
    bvec = ca - n
    cvec = c - ca
    ax = bvec[:, 1:2] * cvec[:, 2:3] - bvec[:, 2:3] * cvec[:, 1:2]
    ay = bvec[:, 2:3] * cvec[:, 0:1] - bvec[:, 0:1] * cvec[:, 2:3]
    az = bvec[:, 0:1] * cvec[:, 1:2] - bvec[:, 1:2] * cvec[:, 0:1]
    a = jnp.concatenate([ax, ay, az], axis=1)
    cb = -0.58273431 * a + 0.56802827 * bvec - 0.54067466 * cvec + ca
    resi = resi_ref[...]
    chain = chain_ref[...]
    zero = jnp.zeros_like(resi)
    table_ref[...] = jnp.concatenate([n, c, ca, cb, resi, chain, zero, zero],
                                     axis=1)
    wp = wp_ref[...] + bp_ref[...]
    t_ref[...] = jnp.dot(wp, we_ref[0:16, :], preferred_element_type=jnp.float32)


def _topk_body(qc_ref, kc_ref, idx_ref, fidx_ref, *, L, TQ):
    b = pl.program_id(0)
    qx = qc_ref[0, :, 0:1]
    qy = qc_ref[0, :, 1:2]
    qz = qc_ref[0, :, 2:3]
    kx = kc_ref[0, 0:1, :]
    ky = kc_ref[0, 1:2, :]
    kz = kc_ref[0, 2:3, :]
    dx = qx - kx
    dy = qy - ky
    dz = qz - kz
    d2 = dx * dx + dy * dy + dz * dz + 1e-6
    iota = lax.broadcasted_iota(jnp.int32, (TQ, L), 1)
    cols = []
    for j in range(K):
        m = jnp.min(d2, axis=1, keepdims=True)
        cand = jnp.where(d2 == m, iota, L)
        idx = jnp.min(cand, axis=1, keepdims=True)
        cols.append(idx)
        if j < K - 1:
            d2 = jnp.where(iota == idx, jnp.inf, d2)
    idxm = jnp.concatenate(cols, axis=1)
    idx_ref[0] = idxm
    fidx_ref[0] = idxm + b * L


def _sc_gather_body(table_hbm, cidx_hbm, out_hbm, idx_v, rows_v, sem,
                    *, n_chunks, chunk):
    wid = lax.axis_index("s") * SC_NC + lax.axis_index("c")
    base = wid * (n_chunks * chunk)
    for ci in range(n_chunks):
        off = base + ci * chunk
        pltpu.sync_copy(cidx_hbm.at[pl.ds(off, chunk)], idx_v)
        pltpu.async_copy(table_hbm.at[idx_v], rows_v, sem).wait()
        pltpu.sync_copy(rows_v, out_hbm.at[pl.ds(off, chunk)])


_PAIRS = [(3, 3), (0, 0), (6, 6), (9, 9), (3, 0), (3, 6), (3, 9), (0, 6),
          (0, 9), (9, 6), (0, 3), (6, 3), (9, 3), (6, 0), (9, 0), (6, 9)]


def _edges_body(gq_ref, gn_ref, t_ref, w2_ref, s_ref, b_ref,
                out_ref, *, EB):
    q = gq_ref[...]
    n = gn_ref[...]
    mu = DMIN + lax.broadcasted_iota(jnp.int32, (NRBF, 1), 0).astype(
        jnp.float32) * ((DMAX - DMIN) / (NRBF - 1))
    sigma = (DMAX - DMIN) / NRBF

    def rbf(d):
        z = (d - mu) / sigma
        return jnp.exp(-(z * z))

    feats = []
    for pq, pn in _PAIRS:
        dx = q[pq:pq + 1, :] - n[pn:pn + 1, :]
        dy = q[pq + 1:pq + 2, :] - n[pn + 1:pn + 2, :]
        dz = q[pq + 2:pq + 3, :] - n[pn + 2:pn + 3, :]
        feats.append(rbf(jnp.sqrt(dx * dx + dy * dy + dz * dz + 1e-6)))
    f = jnp.concatenate(feats, axis=0)
    dims = (((0,), (0,)), ((), ()))
    e = lax.dot_general(f, w2_ref[...], dims,
                        preferred_element_type=jnp.float32)

    off = q[12:13, :] - n[12:13, :]
    same = q[13:14, :] == n[13:14, :]
    d = jnp.where(same, jnp.clip(off + float(MAXREL), 0.0, 2.0 * MAXREL),
                  2.0 * MAXREL + 1.0).astype(jnp.int32)
    onehot = (lax.broadcasted_iota(jnp.int32, (POS_PAD, EB), 0) == d
              ).astype(jnp.float32)
    e = e + lax.dot_general(onehot, t_ref[...], dims,
                            preferred_element_type=jnp.float32)

    m = jnp.mean(e, axis=1, keepdims=True)
    xc = e - m
    var = jnp.mean(xc * xc, axis=1, keepdims=True)
    o = xc * lax.rsqrt(var + 1e-5) * s_ref[...] + b_ref[...]
    out_ref[0] = o.reshape(EB // K, K, 128)


def kernel(X, mask, residue_idx, chain_labels, W_pos, b_pos, W_edge,
           ln_scale, ln_bias):
    B, L = X.shape[0], X.shape[1]
    R = B * L
    E = R * K
    f32 = jnp.float32

    Xr = X.reshape(R, 12)
    resi_f = residue_idx.reshape(R, 1).astype(f32)
    chain_f = chain_labels.reshape(R, 1).astype(f32)
    wp_pad = jnp.zeros((POS_PAD, W_pos.shape[1]), f32).at[:W_pos.shape[0]].set(W_pos)
    bp = b_pos.reshape(1, -1)
    table, T = pl.pallas_call(
        _prep_body,
        out_shape=(jax.ShapeDtypeStruct((R, ROWF), f32),
                   jax.ShapeDtypeStruct((POS_PAD, 128), f32)),
    )(Xr, resi_f, chain_f, wp_pad, bp, W_edge)

    TQ = 256
    Cq = X[:, :, 1, :]
    qc = jnp.concatenate([Cq, jnp.zeros((B, L, 1), f32)], axis=2)
    kc = jnp.swapaxes(qc, 1, 2)
    i32 = jnp.int32
    E_idx, fidx = pl.pallas_call(
        functools.partial(_topk_body, L=L, TQ=TQ),
        grid=(B, L // TQ),
        in_specs=[
            pl.BlockSpec((1, TQ, 4), lambda b, q: (b, q, 0)),
            pl.BlockSpec((1, 4, L), lambda b, q: (b, 0, 0)),
        ],
        out_specs=[
            pl.BlockSpec((1, TQ, K), lambda b, q: (b, q, 0)),
            pl.BlockSpec((1, TQ, K), lambda b, q: (b, q, 0)),
        ],
        out_shape=[
            jax.ShapeDtypeStruct((B, L, K), i32),
            jax.ShapeDtypeStruct((B, L, K), i32),
        ],
    )(qc, kc)

    qidx = jnp.repeat(jnp.arange(R, dtype=i32), K)
    cidx = jnp.concatenate([qidx, fidx.reshape(E)])
    per_w = (2 * E) // SC_NW
    chunk = 1920
    n_chunks = per_w // chunk
    mesh = plsc.VectorSubcoreMesh(core_axis_name="c", subcore_axis_name="s")
    gath = pl.kernel(
        functools.partial(_sc_gather_body, n_chunks=n_chunks, chunk=chunk),
        out_type=jax.ShapeDtypeStruct((2 * E, ROWF), f32),
        mesh=mesh,
        compiler_params=pltpu.CompilerParams(use_tc_tiling_on_sc=False),
        scratch_types=[
            pltpu.VMEM((chunk,), i32),
            pltpu.VMEM((chunk, ROWF), f32),
            pltpu.SemaphoreType.DMA,
        ],
    )(table, cidx)
    gathT = gath.T
    gqT = gathT[:, :E]
    gnT = gathT[:, E:]

    TQ2 = 128
    EB = TQ2 * K
    W2 = W_edge[NRBF:, :]
    Eout = pl.pallas_call(
        functools.partial(_edges_body, EB=EB),
        grid=(B, L // TQ2),
        in_specs=[
            pl.BlockSpec((ROWF, EB), lambda b, i: (0, b * (L // TQ2) + i)),
            pl.BlockSpec((ROWF, EB), lambda b, i: (0, b * (L // TQ2) + i)),
            pl.BlockSpec((POS_PAD, 128), lambda b, i: (0, 0)),
            pl.BlockSpec((W2.shape[0], 128), lambda b, i: (0, 0)),
            pl.BlockSpec((1, 128), lambda b, i: (0, 0)),
            pl.BlockSpec((1, 128), lambda b, i: (0, 0)),
        ],
        out_specs=pl.BlockSpec((1, TQ2, K, 128), lambda b, i: (b, i, 0, 0)),
        out_shape=jax.ShapeDtypeStruct((B, L, K, 128), f32),
    )(gqT, gnT, T, W2, ln_scale.reshape(1, 128), ln_bias.reshape(1, 128))

    return (Eout, E_idx)


# speedup vs baseline: 3.7971x; 1.3617x over previous
import functools

import jax
import jax.numpy as jnp
from jax import lax
from jax.experimental import pallas as pl
from jax.experimental.pallas import tpu as pltpu
from jax.experimental.pallas import tpu_sc as plsc

K = 30
NRBF = 16
MAXREL = 32
DMIN, DMAX = 2.0, 22.0
POS_PAD = 72
ROWF = 16

SC_NC, SC_NS = 2, 16
SC_NW = SC_NC * SC_NS


def _prep_body(x_ref, resi_ref, chain_ref, wp_ref, bp_ref, we_ref,
               table_ref, t_ref):
    x = x_ref[...]
    n = x[:, 0:3]
    c = x[:, 3:6]
    ca = x[:, 6:9]
    bvec = ca - n
    cvec = c - ca
    ax = bvec[:, 1:2] * cvec[:, 2:3] - bvec[:, 2:3] * cvec[:, 1:2]
    ay = bvec[:, 2:3] * cvec[:, 0:1] - bvec[:, 0:1] * cvec[:, 2:3]
    az = bvec[:, 0:1] * cvec[:, 1:2] - bvec[:, 1:2] * cvec[:, 0:1]
    a = jnp.concatenate([ax, ay, az], axis=1)
    cb = -0.58273431 * a + 0.56802827 * bvec - 0.54067466 * cvec + ca
    resi = resi_ref[...]
    chain = chain_ref[...]
    zero = jnp.zeros_like(resi)
    table_ref[...] = jnp.concatenate([n, c, ca, cb, resi, chain, zero, zero],
                                     axis=1)
    wp = wp_ref[...] + bp_ref[...]
    t_ref[...] = jnp.dot(wp, we_ref[0:16, :], preferred_element_type=jnp.float32)


def _topk_body(qc_ref, kc_ref, idx_ref, fidx_ref, *, L, TQ):
    b = pl.program_id(0)
    qx = qc_ref[0, :, 0:1]
    qy = qc_ref[0, :, 1:2]
    qz = qc_ref[0, :, 2:3]
    kx = kc_ref[0, 0:1, :]
    ky = kc_ref[0, 1:2, :]
    kz = kc_ref[0, 2:3, :]
    dx = qx - kx
    dy = qy - ky
    dz = qz - kz
    d2 = dx * dx + dy * dy + dz * dz + 1e-6
    iota_f = lax.broadcasted_iota(jnp.int32, (TQ, L), 1).astype(jnp.float32)
    cols = []
    for j in range(K):
        m = jnp.min(d2, axis=1, keepdims=True)
        cand = jnp.where(d2 == m, iota_f, float(L))
        idxf = jnp.min(cand, axis=1, keepdims=True)
        cols.append(idxf)
        if j < K - 1:
            d2 = jnp.where(iota_f == idxf, jnp.inf, d2)
    idxm = jnp.concatenate(cols, axis=1).astype(jnp.int32)
    idx_ref[0] = idxm
    fidx_ref[0] = idxm + b * L


def _sc_gather_body(table_hbm, cidx_hbm, out_hbm, idx_v, rows_v, sem,
                    *, n_chunks, chunk):
    wid = lax.axis_index("s") * SC_NC + lax.axis_index("c")
    base = wid * (n_chunks * chunk)
    for ci in range(n_chunks):
        off = base + ci * chunk
        pltpu.sync_copy(cidx_hbm.at[pl.ds(off, chunk)], idx_v)
        pltpu.async_copy(table_hbm.at[idx_v], rows_v, sem).wait()
        pltpu.sync_copy(rows_v, out_hbm.at[pl.ds(off, chunk)])


_PAIRS = [(3, 3), (0, 0), (6, 6), (9, 9), (3, 0), (3, 6), (3, 9), (0, 6),
          (0, 9), (9, 6), (0, 3), (6, 3), (9, 3), (6, 0), (9, 0), (6, 9)]


def _edges_body(qt_ref, gn_ref, p_ref, t_ref, w2_ref, s_ref, b_ref,
                out_ref, *, EB):
    dims = (((1,), (0,)), ((), ()))
    q = lax.dot_general(qt_ref[...], p_ref[...], dims,
                        precision=lax.Precision.HIGHEST,
                        preferred_element_type=jnp.float32)
    n = gn_ref[...]
    mu = DMIN + lax.broadcasted_iota(jnp.int32, (NRBF, 1), 0).astype(
        jnp.float32) * ((DMAX - DMIN) / (NRBF - 1))
    sigma = (DMAX - DMIN) / NRBF

    def rbf(d):
        z = (d - mu) / sigma
        return jnp.exp(-(z * z))

    feats = []
    for pq, pn in _PAIRS:
        dx = q[pq:pq + 1, :] - n[pn:pn + 1, :]
        dy = q[pq + 1:pq + 2, :] - n[pn + 1:pn + 2, :]
        dz = q[pq + 2:pq + 3, :] - n[pn + 2:pn + 3, :]
        feats.append(rbf(jnp.sqrt(dx * dx + dy * dy + dz * dz + 1e-6)))
    f = jnp.concatenate(feats, axis=0)
    dims0 = (((0,), (0,)), ((), ()))
    e = lax.dot_general(f, w2_ref[...], dims0,
                        preferred_element_type=jnp.float32)

    off = q[12:13, :] - n[12:13, :]
    same = q[13:14, :] == n[13:14, :]
    d = jnp.where(same, jnp.clip(off + float(MAXREL), 0.0, 2.0 * MAXREL),
                  2.0 * MAXREL + 1.0).astype(jnp.int32)
    onehot = (lax.broadcasted_iota(jnp.int32, (POS_PAD, EB), 0) == d
              ).astype(jnp.float32)
    e = e + lax.dot_general(onehot, t_ref[...], dims0,
                            preferred_element_type=jnp.float32)

    m = jnp.mean(e, axis=1, keepdims=True)
    xc = e - m
    var = jnp.mean(xc * xc, axis=1, keepdims=True)
    o = xc * lax.rsqrt(var + 1e-5) * s_ref[...] + b_ref[...]
    out_ref[0] = o.reshape(EB // K, K, 128)


def kernel(X, mask, residue_idx, chain_labels, W_pos, b_pos, W_edge,
           ln_scale, ln_bias):
    B, L = X.shape[0], X.shape[1]
    R = B * L
    E = R * K
    f32 = jnp.float32

    Xr = X.reshape(R, 12)
    resi_f = residue_idx.reshape(R, 1).astype(f32)
    chain_f = chain_labels.reshape(R, 1).astype(f32)
    wp_pad = jnp.zeros((POS_PAD, W_pos.shape[1]), f32).at[:W_pos.shape[0]].set(W_pos)
    bp = b_pos.reshape(1, -1)
    table, T = pl.pallas_call(
        _prep_body,
        out_shape=(jax.ShapeDtypeStruct((R, ROWF), f32),
                   jax.ShapeDtypeStruct((POS_PAD, 128), f32)),
    )(Xr, resi_f, chain_f, wp_pad, bp, W_edge)

    TQ = 256
    Cq = X[:, :, 1, :]
    qc = jnp.concatenate([Cq, jnp.zeros((B, L, 1), f32)], axis=2)
    kc = jnp.swapaxes(qc, 1, 2)
    i32 = jnp.int32
    E_idx, fidx = pl.pallas_call(
        functools.partial(_topk_body, L=L, TQ=TQ),
        grid=(B, L // TQ),
        in_specs=[
            pl.BlockSpec((1, TQ, 4), lambda b, q: (b, q, 0)),
            pl.BlockSpec((1, 4, L), lambda b, q: (b, 0, 0)),
        ],
        out_specs=[
            pl.BlockSpec((1, TQ, K), lambda b, q: (b, q, 0)),
            pl.BlockSpec((1, TQ, K), lambda b, q: (b, q, 0)),
        ],
        out_shape=[
            jax.ShapeDtypeStruct((B, L, K), i32),
            jax.ShapeDtypeStruct((B, L, K), i32),
        ],
    )(qc, kc)

    cidx = fidx.reshape(E)
    per_w = E // SC_NW
    chunk = 1920
    n_chunks = per_w // chunk
    mesh = plsc.VectorSubcoreMesh(core_axis_name="c", subcore_axis_name="s")
    gath = pl.kernel(
        functools.partial(_sc_gather_body, n_chunks=n_chunks, chunk=chunk),
        out_type=jax.ShapeDtypeStruct((E, ROWF), f32),
        mesh=mesh,
        compiler_params=pltpu.CompilerParams(use_tc_tiling_on_sc=False),
        scratch_types=[
            pltpu.VMEM((chunk,), i32),
            pltpu.VMEM((chunk, ROWF), f32),
            pltpu.SemaphoreType.DMA,
        ],
    )(table, cidx)
    gnT = gath.T
    tableT = table.T

    TQ2 = 128
    EB = TQ2 * K
    W2 = W_edge[NRBF:, :]
    P = jnp.repeat(jnp.eye(TQ2, dtype=f32), K, axis=1)
    Eout = pl.pallas_call(
        functools.partial(_edges_body, EB=EB),
        grid=(B, L // TQ2),
        in_specs=[
            pl.BlockSpec((ROWF, TQ2), lambda b, i: (0, b * (L // TQ2) + i)),
            pl.BlockSpec((ROWF, EB), lambda b, i: (0, b * (L // TQ2) + i)),
            pl.BlockSpec((TQ2, EB), lambda b, i: (0, 0)),
            pl.BlockSpec((POS_PAD, 128), lambda b, i: (0, 0)),
            pl.BlockSpec((W2.shape[0], 128), lambda b, i: (0, 0)),
            pl.BlockSpec((1, 128), lambda b, i: (0, 0)),
            pl.BlockSpec((1, 128), lambda b, i: (0, 0)),
        ],
        out_specs=pl.BlockSpec((1, TQ2, K, 128), lambda b, i: (b, i, 0, 0)),
        out_shape=jax.ShapeDtypeStruct((B, L, K, 128), f32),
    )(tableT, gnT, P, T, W2, ln_scale.reshape(1, 128), ln_bias.reshape(1, 128))

    return (Eout, E_idx)
